# Initial kernel scaffold; baseline (speedup 1.0000x reference)
import jax
import jax.numpy as jnp
from jax.experimental import pallas as pl


def kernel(x, edge_index, W0, a_src0, a_dst0, b0, Wl0, bl0, W1, a_src1, a_dst1, b1, Wl1, bl1, Wmu, a_src_mu, a_dst_mu, b_mu, Wlv, a_src_lv, a_dst_lv, b_lv):
    raise NotImplementedError("write your pallas kernel here")


# speedup vs baseline: 31.4316x; 31.4316x over previous
import functools

import jax
import jax.numpy as jnp
from jax import lax
from jax.experimental import pallas as pl
from jax.experimental.pallas import tpu as pltpu
from jax.experimental.pallas import tpu_sc as plsc

N = 10000
E = 320000
D = 128
DZ = 64

NP = 10240
RPT = NP // 16
NW = 32
EPW = E // NW
K = 80
NCH = EPW // K
EPS = 1e-16


def _leaky(v, slope):
    return jnp.maximum(v, v * slope)


def _make_edge_pass(ngroups):
    gw = D // ngroups

    mesh = plsc.VectorSubcoreMesh(core_axis_name="c", subcore_axis_name="s")

    out_type = [jax.ShapeDtypeStruct((2 * NP, D), jnp.float32)] + [
        jax.ShapeDtypeStruct((2 * NP,), jnp.float32) for _ in range(ngroups)
    ]
    scratch = (
        [pltpu.VMEM((K,), jnp.int32), pltpu.VMEM((K,), jnp.int32)]
        + [pltpu.VMEM((K,), jnp.float32) for _ in range(3 * ngroups)]
        + [
            pltpu.VMEM((K, D), jnp.float32),
            pltpu.VMEM_SHARED((NP, D), jnp.float32),
        ]
        + [pltpu.VMEM_SHARED((NP,), jnp.float32) for _ in range(ngroups)]
        + [pltpu.SemaphoreType.DMA for _ in range(1 + 2 * ngroups)]
    )

    @functools.partial(pl.kernel, out_type=out_type, scratch_types=scratch,
                       mesh=mesh)
    def edge_pass(*refs):
        it = iter(refs)
        h_hbm = next(it)
        a_srcs = [next(it) for _ in range(ngroups)]
        a_dsts = [next(it) for _ in range(ngroups)]
        src_hbm = next(it)
        dst_hbm = next(it)
        zrows_hbm = next(it)
        zs_hbm = next(it)
        acc_out = next(it)
        s_outs = [next(it) for _ in range(ngroups)]
        srcv = next(it)
        dstv = next(it)
        asv = [next(it) for _ in range(ngroups)]
        adv = [next(it) for _ in range(ngroups)]
        pv = [next(it) for _ in range(ngroups)]
        rows = next(it)
        acc_sh = next(it)
        s_shs = [next(it) for _ in range(ngroups)]
        sem_rows = next(it)
        sem_as = [next(it) for _ in range(ngroups)]
        sem_ad = [next(it) for _ in range(ngroups)]

        c = lax.axis_index("c")
        t = lax.axis_index("s")
        wid = t * 2 + c

        pltpu.sync_copy(zrows_hbm.at[pl.ds(t * RPT, RPT)],
                        acc_sh.at[pl.ds(t * RPT, RPT)])
        for g in range(ngroups):
            pltpu.sync_copy(zs_hbm.at[pl.ds(t * RPT, RPT)],
                            s_shs[g].at[pl.ds(t * RPT, RPT)])
        plsc.subcore_barrier()

        def chunk(ci, carry):
            base = wid * EPW + ci * K
            pltpu.sync_copy(src_hbm.at[pl.ds(base, K)], srcv)
            pltpu.sync_copy(dst_hbm.at[pl.ds(base, K)], dstv)
            cps = [pltpu.async_copy(h_hbm.at[srcv], rows, sem_rows)]
            for g in range(ngroups):
                cps.append(pltpu.async_copy(a_srcs[g].at[srcv], asv[g],
                                            sem_as[g]))
                cps.append(pltpu.async_copy(a_dsts[g].at[dstv], adv[g],
                                            sem_ad[g]))
            for cp in cps:
                cp.wait()
            for g in range(ngroups):
                for j in range(K // 16):
                    sl = pl.ds(j * 16, 16)
                    e = asv[g][sl] + adv[g][sl]
                    pv[g][sl] = jnp.exp(jnp.maximum(e, e * 0.2))

            def scale16(j16, carry2):
                base_j = j16 * 16
                p16 = [pv[g][pl.ds(base_j, 16)] for g in range(ngroups)]
                for l in range(16):
                    j = base_j + l
                    for g in range(ngroups):
                        pj = p16[g][l]
                        for cc in range(gw // 16):
                            sl = pl.ds(g * gw + cc * 16, 16)
                            rows[j, sl] = rows[j, sl] * pj
                return carry2

            lax.fori_loop(0, K // 16, scale16, 0)
            pltpu.sync_copy(rows, acc_sh.at[dstv], add=True)
            for g in range(ngroups):
                pltpu.sync_copy(pv[g], s_shs[g].at[dstv], add=True)
            return carry

        lax.fori_loop(0, NCH, chunk, 0)
        plsc.subcore_barrier()

        off = c * NP + t * RPT
        pltpu.sync_copy(acc_sh.at[pl.ds(t * RPT, RPT)],
                        acc_out.at[pl.ds(off, RPT)])
        for g in range(ngroups):
            pltpu.sync_copy(s_shs[g].at[pl.ds(t * RPT, RPT)],
                            s_outs[g].at[pl.ds(off, RPT)])

    return edge_pass


_edge_pass1 = _make_edge_pass(1)
_edge_pass2 = _make_edge_pass(2)


_B = 512
_G = NP // _B


def _row_spec(w):
    return pl.BlockSpec((_B, w), lambda i: (i, 0))


def _full_spec(r, w):
    return pl.BlockSpec((r, w), lambda i: (0, 0))


def _stage_a_body(x_r, W0_r, asv_r, adv_r, Wl0_r, bl0_r,
                  h0_o, as0_o, ad0_o, hl0_o):
    xb = x_r[...]
    h0 = jnp.dot(xb, W0_r[...], preferred_element_type=jnp.float32)
    h0_o[...] = h0
    as0_o[...] = jnp.dot(h0, asv_r[...], preferred_element_type=jnp.float32)
    ad0_o[...] = jnp.dot(h0, adv_r[...], preferred_element_type=jnp.float32)
    hl0_o[...] = (jnp.dot(xb, Wl0_r[...], preferred_element_type=jnp.float32)
                  + bl0_r[...])


def _stage_b_body(acca_r, accb_r, sa_r, sb_r, b0_r, hl0_r,
                  W1_r, asv_r, adv_r, Wl1_r, bl1_r,
                  h1w_o, as1_o, ad1_o, hl1_o):
    g0 = (acca_r[...] + accb_r[...]) / (sa_r[...] + sb_r[...] + EPS) + b0_r[...]
    h1 = _leaky(g0 + hl0_r[...], 0.01)
    h1w = jnp.dot(h1, W1_r[...], preferred_element_type=jnp.float32)
    h1w_o[...] = h1w
    as1_o[...] = jnp.dot(h1w, asv_r[...], preferred_element_type=jnp.float32)
    ad1_o[...] = jnp.dot(h1w, adv_r[...], preferred_element_type=jnp.float32)
    hl1_o[...] = (jnp.dot(h1, Wl1_r[...], preferred_element_type=jnp.float32)
                  + bl1_r[...])


def _stage_c_body(acca_r, accb_r, sa_r, sb_r, as1_r, ad1_r, h1w_r, hl1_r,
                  b1_r, Wmu_r, asmu_r, admu_r, Wlv_r, aslv_r, adlv_r,
                  hml_o, asmu_o, admu_o, aslv_o, adlv_o):
    e1 = as1_r[...] + ad1_r[...]
    sp = jnp.exp(_leaky(e1, 0.2))
    den = sa_r[...] + sb_r[...] + sp + EPS
    g1 = (acca_r[...] + accb_r[...] + sp * h1w_r[...]) / den + b1_r[...]
    h2 = _leaky(g1 + hl1_r[...], 0.01)
    hmu = jnp.dot(h2, Wmu_r[...], preferred_element_type=jnp.float32)
    hlv = jnp.dot(h2, Wlv_r[...], preferred_element_type=jnp.float32)
    hml_o[...] = jnp.concatenate([hmu, hlv], axis=1)
    asmu_o[...] = jnp.dot(hmu, asmu_r[...], preferred_element_type=jnp.float32)
    admu_o[...] = jnp.dot(hmu, admu_r[...], preferred_element_type=jnp.float32)
    aslv_o[...] = jnp.dot(hlv, aslv_r[...], preferred_element_type=jnp.float32)
    adlv_o[...] = jnp.dot(hlv, adlv_r[...], preferred_element_type=jnp.float32)


def _stage_d_body(acca_r, accb_r, smua_r, smub_r, slva_r, slvb_r,
                  asmu_r, admu_r, aslv_r, adlv_r, hml_r, bmu_r, blv_r,
                  mu_o, lv_o):
    spmu = jnp.exp(_leaky(asmu_r[...] + admu_r[...], 0.2))
    splv = jnp.exp(_leaky(aslv_r[...] + adlv_r[...], 0.2))
    acc = acca_r[...] + accb_r[...]
    hml = hml_r[...]
    mu_o[...] = ((acc[:, :DZ] + spmu * hml[:, :DZ])
                 / (smua_r[...] + smub_r[...] + spmu + EPS) + bmu_r[...])
    lv_o[...] = ((acc[:, DZ:] + splv * hml[:, DZ:])
                 / (slva_r[...] + slvb_r[...] + splv + EPS) + blv_r[...])


def _stage_a(x_p, W0, a_src0, a_dst0, Wl0, bl0):
    return pl.pallas_call(
        _stage_a_body,
        grid=(_G,),
        in_specs=[_row_spec(D), _full_spec(D, D), _full_spec(D, 1),
                  _full_spec(D, 1), _full_spec(D, D), _full_spec(1, D)],
        out_specs=[_row_spec(D), _row_spec(1), _row_spec(1), _row_spec(D)],
        out_shape=[jax.ShapeDtypeStruct((NP, D), jnp.float32),
                   jax.ShapeDtypeStruct((NP, 1), jnp.float32),
                   jax.ShapeDtypeStruct((NP, 1), jnp.float32),
                   jax.ShapeDtypeStruct((NP, D), jnp.float32)],
    )(x_p, W0, a_src0.reshape(D, 1), a_dst0.reshape(D, 1), Wl0,
      bl0.reshape(1, D))


def _stage_b(acca, accb, sa, sb, b0, hl0, W1, a_src1, a_dst1, Wl1, bl1):
    return pl.pallas_call(
        _stage_b_body,
        grid=(_G,),
        in_specs=[_row_spec(D), _row_spec(D), _row_spec(1), _row_spec(1),
                  _full_spec(1, D), _row_spec(D), _full_spec(D, D),
                  _full_spec(D, 1), _full_spec(D, 1), _full_spec(D, D),
                  _full_spec(1, D)],
        out_specs=[_row_spec(D), _row_spec(1), _row_spec(1), _row_spec(D)],
        out_shape=[jax.ShapeDtypeStruct((NP, D), jnp.float32),
                   jax.ShapeDtypeStruct((NP, 1), jnp.float32),
                   jax.ShapeDtypeStruct((NP, 1), jnp.float32),
                   jax.ShapeDtypeStruct((NP, D), jnp.float32)],
    )(acca, accb, sa, sb, b0.reshape(1, D), hl0, W1,
      a_src1.reshape(D, 1), a_dst1.reshape(D, 1), Wl1, bl1.reshape(1, D))


def _stage_c(acca, accb, sa, sb, as1, ad1, h1w, hl1, b1,
             Wmu, a_src_mu, a_dst_mu, Wlv, a_src_lv, a_dst_lv):
    return pl.pallas_call(
        _stage_c_body,
        grid=(_G,),
        in_specs=[_row_spec(D), _row_spec(D), _row_spec(1), _row_spec(1),
                  _row_spec(1), _row_spec(1), _row_spec(D), _row_spec(D),
                  _full_spec(1, D), _full_spec(D, DZ), _full_spec(DZ, 1),
                  _full_spec(DZ, 1), _full_spec(D, DZ), _full_spec(DZ, 1),
                  _full_spec(DZ, 1)],
        out_specs=[_row_spec(D), _row_spec(1), _row_spec(1), _row_spec(1),
                   _row_spec(1)],
        out_shape=[jax.ShapeDtypeStruct((NP, D), jnp.float32)] +
                  [jax.ShapeDtypeStruct((NP, 1), jnp.float32)] * 4,
    )(acca, accb, sa, sb, as1, ad1, h1w, hl1, b1.reshape(1, D),
      Wmu, a_src_mu.reshape(DZ, 1), a_dst_mu.reshape(DZ, 1),
      Wlv, a_src_lv.reshape(DZ, 1), a_dst_lv.reshape(DZ, 1))


def _stage_d(acca, accb, smua, smub, slva, slvb, asmu, admu, aslv, adlv,
             hml, b_mu, b_lv):
    return pl.pallas_call(
        _stage_d_body,
        grid=(_G,),
        in_specs=[_row_spec(D), _row_spec(D)] + [_row_spec(1)] * 4 +
                 [_row_spec(1)] * 4 + [_row_spec(D), _full_spec(1, DZ),
                                       _full_spec(1, DZ)],
        out_specs=[_row_spec(DZ), _row_spec(DZ)],
        out_shape=[jax.ShapeDtypeStruct((NP, DZ), jnp.float32),
                   jax.ShapeDtypeStruct((NP, DZ), jnp.float32)],
    )(acca, accb, smua, smub, slva, slvb, asmu, admu, aslv, adlv, hml,
      b_mu.reshape(1, DZ), b_lv.reshape(1, DZ))


def kernel(x, edge_index, W0, a_src0, a_dst0, b0, Wl0, bl0,
           W1, a_src1, a_dst1, b1, Wl1, bl1,
           Wmu, a_src_mu, a_dst_mu, b_mu,
           Wlv, a_src_lv, a_dst_lv, b_lv):
    src = edge_index[0].astype(jnp.int32)
    dst = edge_index[1].astype(jnp.int32)
    x_p = jnp.pad(x, ((0, NP - N), (0, 0)))
    zrows = jnp.zeros((NP, D), jnp.float32)
    zs = jnp.zeros((NP,), jnp.float32)

    h0, as0, ad0, hl0 = _stage_a(x_p, W0, a_src0, a_dst0, Wl0, bl0)

    acc0, s0 = _edge_pass1(h0, as0.reshape(NP), ad0.reshape(NP), src, dst,
                           zrows, zs)
    acc0a, acc0b = acc0[:NP], acc0[NP:]
    s0a, s0b = s0[:NP].reshape(NP, 1), s0[NP:].reshape(NP, 1)

    h1w, as1, ad1, hl1 = _stage_b(acc0a, acc0b, s0a, s0b, b0, hl0,
                                  W1, a_src1, a_dst1, Wl1, bl1)

    acc1, s1 = _edge_pass1(h1w, as1.reshape(NP), ad1.reshape(NP), src, dst,
                           zrows, zs)
    acc1a, acc1b = acc1[:NP], acc1[NP:]
    s1a, s1b = s1[:NP].reshape(NP, 1), s1[NP:].reshape(NP, 1)

    hml, asmu, admu, aslv, adlv = _stage_c(
        acc1a, acc1b, s1a, s1b, as1, ad1, h1w, hl1, b1,
        Wmu, a_src_mu, a_dst_mu, Wlv, a_src_lv, a_dst_lv)

    acc2, smu, slv = _edge_pass2(
        hml, asmu.reshape(NP), aslv.reshape(NP), admu.reshape(NP),
        adlv.reshape(NP), src, dst, zrows, zs)
    acc2a, acc2b = acc2[:NP], acc2[NP:]

    mu, lv = _stage_d(
        acc2a, acc2b,
        smu[:NP].reshape(NP, 1), smu[NP:].reshape(NP, 1),
        slv[:NP].reshape(NP, 1), slv[NP:].reshape(NP, 1),
        asmu, admu, aslv, adlv, hml, b_mu, b_lv)

    return (mu[:N], lv[:N])
